# Initial kernel scaffold; baseline (speedup 1.0000x reference)
import jax
import jax.numpy as jnp
from jax.experimental import pallas as pl


def kernel(users, adjacent_items, items_pool, items_weight, edge_user, edge_item, user_preference, item_preference, user_map, item_map):
    raise NotImplementedError("write your pallas kernel here")


# speedup vs baseline: 55.2609x; 55.2609x over previous
import functools

import jax
import jax.numpy as jnp
from jax import lax
from jax.experimental import pallas as pl
from jax.experimental.pallas import tpu as pltpu
from jax.experimental.pallas import tpu_sc as plsc

NU = 50000
NI = 50000
EMB = 64
NE = 800000
NE_PAD = 819200
BATCH = 4096
POOL = 10

NC = 2
NS = 16
NW = NC * NS

E1 = NE_PAD // NW
EB1 = 400

ES = NE_PAD // NS
BS = 1280
CK = 128

HALF = NU // 2
SPR = 25600
DRAIN = SPR // NS

_mesh = plsc.VectorSubcoreMesh(
    core_axis_name="c", subcore_axis_name="s", num_cores=NC, num_subcores=NS)
_sc_params = pltpu.CompilerParams(
    needs_layout_passes=False, use_tc_tiling_on_sc=False)

_f32 = jnp.float32
_i32 = jnp.int32


def _k1_body(eu_hbm, ei_hbm, skey_hbm, mu_hbm, mi_hbm,
             skey_v, eu_s, ei_s, mu_s, mi_s):
    c = lax.axis_index("c")
    s = lax.axis_index("s")
    wid = s * NC + c
    base = wid * E1
    pltpu.sync_copy(skey_hbm, skey_v)

    def block(b, carry):
        off = base + b * EB1
        pltpu.sync_copy(eu_hbm.at[pl.ds(off, EB1)], eu_s)
        pltpu.sync_copy(ei_hbm.at[pl.ds(off, EB1)], ei_s)
        for v in range(EB1 // 16):
            euv = eu_s[pl.ds(v * 16, 16)]
            eiv = ei_s[pl.ds(v * 16, 16)]
            ek = (euv * NI + eiv) ^ jnp.int32(-2147483648)
            lo = jnp.zeros((16,), _i32)
            for sh in range(11, -1, -1):
                probe = lo + ((1 << sh) - 1)
                kv = plsc.load_gather(skey_v, [probe])
                lo = lo + jnp.where(kv < ek, _i32(1 << sh), _i32(0))
            kv = plsc.load_gather(skey_v, [lo])
            pos = off + v * 16 + lax.iota(_i32, 16)
            drop = (kv == ek) | (pos >= NE)
            mu_s[pl.ds(v * 16, 16)] = jnp.where(drop, -1, euv)
            mi_s[pl.ds(v * 16, 16)] = jnp.where(drop, -1, eiv)
        pltpu.sync_copy(mu_s, mu_hbm.at[pl.ds(off, EB1)])
        pltpu.sync_copy(mi_s, mi_hbm.at[pl.ds(off, EB1)])
        return carry

    lax.fori_loop(_i32(0), _i32(E1 // EB1), block, _i32(0))


_k1 = pl.kernel(
    _k1_body,
    out_type=(jax.ShapeDtypeStruct((NE_PAD,), _i32),
              jax.ShapeDtypeStruct((NE_PAD,), _i32)),
    mesh=_mesh,
    scratch_types=[
        pltpu.VMEM((BATCH,), _i32),
        pltpu.VMEM((EB1,), _i32),
        pltpu.VMEM((EB1,), _i32),
        pltpu.VMEM((EB1,), _i32),
        pltpu.VMEM((EB1,), _i32),
    ],
    compiler_params=_sc_params,
)


def _seg_body(src_hbm, dst_hbm, table_hbm, zeros_hbm, out_hbm,
              src_s, dst_s, gidx, sidx, rows, sp, gsem):
    c = lax.axis_index("c")
    s = lax.axis_index("s")
    pltpu.sync_copy(zeros_hbm, sp.at[pl.ds(s * DRAIN, DRAIN)])
    plsc.subcore_barrier()

    def block(b, carry):
        off = s * ES + b * BS
        pltpu.sync_copy(src_hbm.at[pl.ds(off, BS)], src_s)
        pltpu.sync_copy(dst_hbm.at[pl.ds(off, BS)], dst_s)
        for j in range(BS // CK):
            for v in range(CK // 16):
                sv = src_s[pl.ds(j * CK + v * 16, 16)]
                gidx[pl.ds(v * 16, 16)] = sv
                dv = dst_s[pl.ds(j * CK + v * 16, 16)]
                ld = dv - c * HALF
                owned = (ld >= 0) & (ld < HALF)
                tr = HALF + ((j * CK + v * 16 + lax.iota(_i32, 16)) & 511)
                sidx[pl.ds(v * 16, 16)] = jnp.where(owned, ld, tr)
            pltpu.async_copy(table_hbm.at[gidx], rows, gsem).wait()
            pltpu.sync_copy(rows, sp.at[sidx], add=True)
        return carry

    lax.fori_loop(_i32(0), _i32(ES // BS), block, _i32(0))
    plsc.subcore_barrier()

    @pl.when(s < NS - 1)
    def _():
        pltpu.sync_copy(sp.at[pl.ds(s * DRAIN, DRAIN)],
                        out_hbm.at[pl.ds(c * HALF + s * DRAIN, DRAIN)])

    @pl.when(s == NS - 1)
    def _():
        rem = HALF - (NS - 1) * DRAIN
        pltpu.sync_copy(sp.at[pl.ds((NS - 1) * DRAIN, rem)],
                        out_hbm.at[pl.ds(c * HALF + (NS - 1) * DRAIN, rem)])


_seg = pl.kernel(
    _seg_body,
    out_type=jax.ShapeDtypeStruct((NU, EMB), _f32),
    mesh=_mesh,
    scratch_types=[
        pltpu.VMEM((BS,), _i32),
        pltpu.VMEM((BS,), _i32),
        pltpu.VMEM((CK,), _i32),
        pltpu.VMEM((CK,), _i32),
        pltpu.VMEM((CK, EMB), _f32),
        pltpu.VMEM_SHARED((SPR, EMB), _f32),
        pltpu.SemaphoreType.DMA,
    ],
    compiler_params=_sc_params,
)


def _combine_chunk(idx_hbm, base, t0_hbm, t1_hbm, t2_hbm, tm_hbm,
                   out_c_hbm, out_m_hbm, idxb, bufa, bufb, bufc, bufo, gsem):
    pltpu.sync_copy(idx_hbm.at[pl.ds(base, CK)], idxb)
    pltpu.async_copy(t0_hbm.at[idxb], bufa, gsem).wait()
    pltpu.async_copy(t1_hbm.at[idxb], bufb, gsem).wait()
    pltpu.async_copy(t2_hbm.at[idxb], bufc, gsem).wait()

    def row(r, carry):
        for cv in range(EMB // 16):
            a = bufa[r, pl.ds(cv * 16, 16)]
            b = bufb[r, pl.ds(cv * 16, 16)]
            cc = bufc[r, pl.ds(cv * 16, 16)]
            t = (a + (b + cc * (1.0 / 16.0)) * (1.0 / 16.0)) * (1.0 / 3.0)
            bufo[r, pl.ds(cv * 16, 16)] = t
        return carry

    lax.fori_loop(_i32(0), _i32(CK), row, _i32(0))
    pltpu.sync_copy(bufo, out_c_hbm.at[pl.ds(base, CK)])
    pltpu.async_copy(tm_hbm.at[idxb], bufa, gsem).wait()
    pltpu.sync_copy(bufa, out_m_hbm.at[pl.ds(base, CK)])


def _k3_body(u0, u1t, u2t, umap, i0, i1t, i2t, imap, uidx, iidx,
             uc, ug, icm, ig, idxb, bufa, bufb, bufc, bufo, gsem):
    c = lax.axis_index("c")
    s = lax.axis_index("s")
    wid = s * NC + c
    _combine_chunk(uidx, wid * CK, u0, u1t, u2t, umap, uc, ug,
                   idxb, bufa, bufb, bufc, bufo, gsem)
    for q in range(POOL):
        _combine_chunk(iidx, wid * (POOL * CK) + q * CK, i0, i1t, i2t, imap,
                       icm, ig, idxb, bufa, bufb, bufc, bufo, gsem)


_k3 = pl.kernel(
    _k3_body,
    out_type=(jax.ShapeDtypeStruct((BATCH, EMB), _f32),
              jax.ShapeDtypeStruct((BATCH, EMB), _f32),
              jax.ShapeDtypeStruct((BATCH * POOL, EMB), _f32),
              jax.ShapeDtypeStruct((BATCH * POOL, EMB), _f32)),
    mesh=_mesh,
    scratch_types=[
        pltpu.VMEM((CK,), _i32),
        pltpu.VMEM((CK, EMB), _f32),
        pltpu.VMEM((CK, EMB), _f32),
        pltpu.VMEM((CK, EMB), _f32),
        pltpu.VMEM((CK, EMB), _f32),
        pltpu.SemaphoreType.DMA,
    ],
    compiler_params=_sc_params,
)

_SB = 512


def _score_body(uc_ref, ug_ref, ic_ref, ig_ref, w_ref, o_ref):
    uc = uc_ref[...]
    ug = ug_ref[...]
    w = w_ref[...]
    cols = []
    for p in range(POOL):
        sdot = jnp.sum(uc * ic_ref[p] + ug * ig_ref[p], axis=1)
        rating = 1.0 / (1.0 + jnp.exp(-sdot))
        cols.append(0.5 * w[:, p] + rating)
    o_ref[...] = jnp.stack(cols, axis=1)


def _score(uc, ug, ic3, ig3, w):
    return pl.pallas_call(
        _score_body,
        out_shape=jax.ShapeDtypeStruct((BATCH, POOL), _f32),
        grid=(BATCH // _SB,),
        in_specs=[
            pl.BlockSpec((_SB, EMB), lambda b: (b, _i32(0))),
            pl.BlockSpec((_SB, EMB), lambda b: (b, _i32(0))),
            pl.BlockSpec((POOL, _SB, EMB), lambda b: (_i32(0), b, _i32(0))),
            pl.BlockSpec((POOL, _SB, EMB), lambda b: (_i32(0), b, _i32(0))),
            pl.BlockSpec((_SB, POOL), lambda b: (b, _i32(0))),
        ],
        out_specs=pl.BlockSpec((_SB, POOL), lambda b: (b, _i32(0))),
    )(uc, ug, ic3, ig3, w)


def kernel(users, adjacent_items, items_pool, items_weight, edge_user,
           edge_item, user_preference, item_preference, user_map, item_map):
    eu = edge_user.astype(_i32)
    ei = edge_item.astype(_i32)
    pad_src = jnp.arange(NE_PAD - NE, dtype=_i32) % NU
    eu_p = jnp.concatenate([eu, pad_src])
    ei_p = jnp.concatenate([ei, pad_src])

    k_u32 = (users.astype(jnp.uint32) * jnp.uint32(NI)
             + adjacent_items.astype(jnp.uint32))
    skey = lax.bitcast_convert_type(
        jnp.sort(k_u32) ^ jnp.uint32(0x80000000), _i32)

    mu, mi = _k1(eu_p, ei_p, skey)

    up = user_preference.astype(_f32)
    ip = item_preference.astype(_f32)
    zeros = jnp.zeros((DRAIN, EMB), _f32)
    u1 = _seg(ei_p, mu, ip, zeros)
    i1 = _seg(eu_p, mi, up, zeros)
    u2 = _seg(ei_p, mu, i1, zeros)
    i2 = _seg(eu_p, mi, u1, zeros)

    uidx = users.astype(_i32)
    iidx = items_pool.astype(_i32).T.reshape(-1)
    uc, ug, icm, ig = _k3(up, u1, u2, user_map.astype(_f32),
                          ip, i1, i2, item_map.astype(_f32), uidx, iidx)

    ic3 = icm.reshape(POOL, BATCH, EMB)
    ig3 = ig.reshape(POOL, BATCH, EMB)
    return _score(uc, ug, ic3, ig3, items_weight.astype(_f32))


# speedup vs baseline: 83.7423x; 1.5154x over previous
import functools

import jax
import jax.numpy as jnp
from jax import lax
from jax.experimental import pallas as pl
from jax.experimental.pallas import tpu as pltpu
from jax.experimental.pallas import tpu_sc as plsc

NU = 50000
NI = 50000
EMB = 64
NE = 800000
NE_PAD = 819200
BATCH = 4096
POOL = 10

NC = 2
NS = 16
NW = NC * NS

E1 = NE_PAD // NW
EB1 = 400

ES = NE_PAD // NS
BS = 2560
CK = 128

HALF = NU // 2
SPR = 25600
DRAIN = SPR // NS

_mesh = plsc.VectorSubcoreMesh(
    core_axis_name="c", subcore_axis_name="s", num_cores=NC, num_subcores=NS)
_sc_params = pltpu.CompilerParams(
    needs_layout_passes=False, use_tc_tiling_on_sc=False)

_f32 = jnp.float32
_i32 = jnp.int32


def _k1_body(eu_hbm, ei_hbm, skey_hbm, mu_hbm, mi_hbm,
             skey_v, eu_s, ei_s, mu_s, mi_s):
    c = lax.axis_index("c")
    s = lax.axis_index("s")
    wid = s * NC + c
    base = wid * E1
    pltpu.sync_copy(skey_hbm, skey_v)

    def block(b, carry):
        off = base + b * EB1
        pltpu.sync_copy(eu_hbm.at[pl.ds(off, EB1)], eu_s)
        pltpu.sync_copy(ei_hbm.at[pl.ds(off, EB1)], ei_s)
        for v in range(EB1 // 16):
            euv = eu_s[pl.ds(v * 16, 16)]
            eiv = ei_s[pl.ds(v * 16, 16)]
            ek = (euv * NI + eiv) ^ jnp.int32(-2147483648)
            lo = jnp.zeros((16,), _i32)
            for sh in range(11, -1, -1):
                probe = lo + ((1 << sh) - 1)
                kv = plsc.load_gather(skey_v, [probe])
                lo = lo + jnp.where(kv < ek, _i32(1 << sh), _i32(0))
            kv = plsc.load_gather(skey_v, [lo])
            pos = off + v * 16 + lax.iota(_i32, 16)
            drop = (kv == ek) | (pos >= NE)
            mu_s[pl.ds(v * 16, 16)] = jnp.where(drop, -1, euv)
            mi_s[pl.ds(v * 16, 16)] = jnp.where(drop, -1, eiv)
        pltpu.sync_copy(mu_s, mu_hbm.at[pl.ds(off, EB1)])
        pltpu.sync_copy(mi_s, mi_hbm.at[pl.ds(off, EB1)])
        return carry

    lax.fori_loop(_i32(0), _i32(E1 // EB1), block, _i32(0))


_k1 = pl.kernel(
    _k1_body,
    out_type=(jax.ShapeDtypeStruct((NE_PAD,), _i32),
              jax.ShapeDtypeStruct((NE_PAD,), _i32)),
    mesh=_mesh,
    scratch_types=[
        pltpu.VMEM((BATCH,), _i32),
        pltpu.VMEM((EB1,), _i32),
        pltpu.VMEM((EB1,), _i32),
        pltpu.VMEM((EB1,), _i32),
        pltpu.VMEM((EB1,), _i32),
    ],
    compiler_params=_sc_params,
)


def _seg_body(src_hbm, dst_hbm, table_hbm, zeros_hbm, out_hbm,
              src_s, dst_s, gidx0, gidx1, sidx0, sidx1, rows0, rows1,
              sp, gsem0, gsem1, ssem0, ssem1):
    c = lax.axis_index("c")
    s = lax.axis_index("s")
    gidx = (gidx0, gidx1)
    sidx = (sidx0, sidx1)
    rows = (rows0, rows1)
    gsem = (gsem0, gsem1)
    ssem = (ssem0, ssem1)
    pltpu.sync_copy(zeros_hbm, sp.at[pl.ds(s * DRAIN, DRAIN)])
    plsc.subcore_barrier()
    n_chunks = BS // CK

    def block(b, carry):
        off = s * ES + b * BS
        pltpu.sync_copy(src_hbm.at[pl.ds(off, BS)], src_s)
        pltpu.sync_copy(dst_hbm.at[pl.ds(off, BS)], dst_s)

        def make_idx(j):
            bb = j & 1
            for v in range(CK // 16):
                sv = src_s[pl.ds(j * CK + v * 16, 16)]
                gidx[bb][pl.ds(v * 16, 16)] = sv
                dv = dst_s[pl.ds(j * CK + v * 16, 16)]
                ld = dv - c * HALF
                owned = (ld >= 0) & (ld < HALF)
                tr = HALF + ((j * CK + v * 16 + lax.iota(_i32, 16)) & 511)
                sidx[bb][pl.ds(v * 16, 16)] = jnp.where(owned, ld, tr)

        gd = [None, None]
        sd = [None, None]
        make_idx(0)
        gd[0] = pltpu.async_copy(table_hbm.at[gidx[0]], rows[0], gsem[0])
        for j in range(n_chunks):
            bb = j & 1
            nb = 1 - bb
            if j + 1 < n_chunks:
                if sd[nb] is not None:
                    sd[nb].wait()
                make_idx(j + 1)
                gd[nb] = pltpu.async_copy(table_hbm.at[gidx[nb]],
                                          rows[nb], gsem[nb])
            gd[bb].wait()
            sd[bb] = pltpu.async_copy(rows[bb], sp.at[sidx[bb]],
                                      ssem[bb], add=True)
        sd[0].wait()
        sd[1].wait()
        return carry

    lax.fori_loop(_i32(0), _i32(ES // BS), block, _i32(0))
    plsc.subcore_barrier()

    @pl.when(s < NS - 1)
    def _():
        pltpu.sync_copy(sp.at[pl.ds(s * DRAIN, DRAIN)],
                        out_hbm.at[pl.ds(c * HALF + s * DRAIN, DRAIN)])

    @pl.when(s == NS - 1)
    def _():
        rem = HALF - (NS - 1) * DRAIN
        pltpu.sync_copy(sp.at[pl.ds((NS - 1) * DRAIN, rem)],
                        out_hbm.at[pl.ds(c * HALF + (NS - 1) * DRAIN, rem)])


_seg = pl.kernel(
    _seg_body,
    out_type=jax.ShapeDtypeStruct((NU, EMB), _f32),
    mesh=_mesh,
    scratch_types=[
        pltpu.VMEM((BS,), _i32),
        pltpu.VMEM((BS,), _i32),
        pltpu.VMEM((CK,), _i32),
        pltpu.VMEM((CK,), _i32),
        pltpu.VMEM((CK,), _i32),
        pltpu.VMEM((CK,), _i32),
        pltpu.VMEM((CK, EMB), _f32),
        pltpu.VMEM((CK, EMB), _f32),
        pltpu.VMEM_SHARED((SPR, EMB), _f32),
        pltpu.SemaphoreType.DMA,
        pltpu.SemaphoreType.DMA,
        pltpu.SemaphoreType.DMA,
        pltpu.SemaphoreType.DMA,
    ],
    compiler_params=_sc_params,
)


def _combine_chunk(idx_hbm, base, t0_hbm, t1_hbm, t2_hbm, tm_hbm,
                   out_c_hbm, out_m_hbm, idxb, bufa, bufb, bufc, bufo, gsem):
    pltpu.sync_copy(idx_hbm.at[pl.ds(base, CK)], idxb)
    pltpu.async_copy(t0_hbm.at[idxb], bufa, gsem).wait()
    pltpu.async_copy(t1_hbm.at[idxb], bufb, gsem).wait()
    pltpu.async_copy(t2_hbm.at[idxb], bufc, gsem).wait()

    def row(r, carry):
        for cv in range(EMB // 16):
            a = bufa[r, pl.ds(cv * 16, 16)]
            b = bufb[r, pl.ds(cv * 16, 16)]
            cc = bufc[r, pl.ds(cv * 16, 16)]
            t = (a + (b + cc * (1.0 / 16.0)) * (1.0 / 16.0)) * (1.0 / 3.0)
            bufo[r, pl.ds(cv * 16, 16)] = t
        return carry

    lax.fori_loop(_i32(0), _i32(CK), row, _i32(0))
    pltpu.sync_copy(bufo, out_c_hbm.at[pl.ds(base, CK)])
    pltpu.async_copy(tm_hbm.at[idxb], bufa, gsem).wait()
    pltpu.sync_copy(bufa, out_m_hbm.at[pl.ds(base, CK)])


def _k3_body(u0, u1t, u2t, umap, i0, i1t, i2t, imap, uidx, iidx,
             uc, ug, icm, ig, idxb, bufa, bufb, bufc, bufo, gsem):
    c = lax.axis_index("c")
    s = lax.axis_index("s")
    wid = s * NC + c
    _combine_chunk(uidx, wid * CK, u0, u1t, u2t, umap, uc, ug,
                   idxb, bufa, bufb, bufc, bufo, gsem)
    for q in range(POOL):
        _combine_chunk(iidx, wid * (POOL * CK) + q * CK, i0, i1t, i2t, imap,
                       icm, ig, idxb, bufa, bufb, bufc, bufo, gsem)


_k3 = pl.kernel(
    _k3_body,
    out_type=(jax.ShapeDtypeStruct((BATCH, EMB), _f32),
              jax.ShapeDtypeStruct((BATCH, EMB), _f32),
              jax.ShapeDtypeStruct((BATCH * POOL, EMB), _f32),
              jax.ShapeDtypeStruct((BATCH * POOL, EMB), _f32)),
    mesh=_mesh,
    scratch_types=[
        pltpu.VMEM((CK,), _i32),
        pltpu.VMEM((CK, EMB), _f32),
        pltpu.VMEM((CK, EMB), _f32),
        pltpu.VMEM((CK, EMB), _f32),
        pltpu.VMEM((CK, EMB), _f32),
        pltpu.SemaphoreType.DMA,
    ],
    compiler_params=_sc_params,
)

_SB = 512


def _score_body(uc_ref, ug_ref, ic_ref, ig_ref, w_ref, o_ref):
    uc = uc_ref[...]
    ug = ug_ref[...]
    w = w_ref[...]
    cols = []
    for p in range(POOL):
        sdot = jnp.sum(uc * ic_ref[p] + ug * ig_ref[p], axis=1)
        rating = 1.0 / (1.0 + jnp.exp(-sdot))
        cols.append(0.5 * w[:, p] + rating)
    o_ref[...] = jnp.stack(cols, axis=1)


def _score(uc, ug, ic3, ig3, w):
    return pl.pallas_call(
        _score_body,
        out_shape=jax.ShapeDtypeStruct((BATCH, POOL), _f32),
        grid=(BATCH // _SB,),
        in_specs=[
            pl.BlockSpec((_SB, EMB), lambda b: (b, _i32(0))),
            pl.BlockSpec((_SB, EMB), lambda b: (b, _i32(0))),
            pl.BlockSpec((POOL, _SB, EMB), lambda b: (_i32(0), b, _i32(0))),
            pl.BlockSpec((POOL, _SB, EMB), lambda b: (_i32(0), b, _i32(0))),
            pl.BlockSpec((_SB, POOL), lambda b: (b, _i32(0))),
        ],
        out_specs=pl.BlockSpec((_SB, POOL), lambda b: (b, _i32(0))),
    )(uc, ug, ic3, ig3, w)


def kernel(users, adjacent_items, items_pool, items_weight, edge_user,
           edge_item, user_preference, item_preference, user_map, item_map):
    eu = edge_user.astype(_i32)
    ei = edge_item.astype(_i32)
    pad_src = jnp.arange(NE_PAD - NE, dtype=_i32) % NU
    eu_p = jnp.concatenate([eu, pad_src])
    ei_p = jnp.concatenate([ei, pad_src])

    k_u32 = (users.astype(jnp.uint32) * jnp.uint32(NI)
             + adjacent_items.astype(jnp.uint32))
    skey = lax.bitcast_convert_type(
        jnp.sort(k_u32) ^ jnp.uint32(0x80000000), _i32)

    mu, mi = _k1(eu_p, ei_p, skey)

    up = user_preference.astype(_f32)
    ip = item_preference.astype(_f32)
    zeros = jnp.zeros((DRAIN, EMB), _f32)
    u1 = _seg(ei_p, mu, ip, zeros)
    i1 = _seg(eu_p, mi, up, zeros)
    u2 = _seg(ei_p, mu, i1, zeros)
    i2 = _seg(eu_p, mi, u1, zeros)

    uidx = users.astype(_i32)
    iidx = items_pool.astype(_i32).T.reshape(-1)
    uc, ug, icm, ig = _k3(up, u1, u2, user_map.astype(_f32),
                          ip, i1, i2, item_map.astype(_f32), uidx, iidx)

    ic3 = icm.reshape(POOL, BATCH, EMB)
    ig3 = ig.reshape(POOL, BATCH, EMB)
    return _score(uc, ug, ic3, ig3, items_weight.astype(_f32))


# speedup vs baseline: 94.3336x; 1.1265x over previous
import functools

import jax
import jax.numpy as jnp
from jax import lax
from jax.experimental import pallas as pl
from jax.experimental.pallas import tpu as pltpu
from jax.experimental.pallas import tpu_sc as plsc

NU = 50000
NI = 50000
EMB = 64
NE = 800000
NE_PAD = 819200
BATCH = 4096
POOL = 10

NC = 2
NS = 16
NW = NC * NS

E1 = NE_PAD // NW
EB1 = 800

ES = NE_PAD // NS
BS = 2048
CK = 128

HALF = NU // 2
SPR = 25152
DRAIN = SPR // NS

_mesh = plsc.VectorSubcoreMesh(
    core_axis_name="c", subcore_axis_name="s", num_cores=NC, num_subcores=NS)
_sc_params = pltpu.CompilerParams(
    needs_layout_passes=False, use_tc_tiling_on_sc=False)

_f32 = jnp.float32
_i32 = jnp.int32


def _k1_body(eu_hbm, ei_hbm, skey_hbm, mu_hbm, mi_hbm,
             skey_v, eu_s, ei_s, mu_s, mi_s):
    c = lax.axis_index("c")
    s = lax.axis_index("s")
    wid = s * NC + c
    base = wid * E1
    pltpu.sync_copy(skey_hbm, skey_v)

    def block(b, carry):
        off = base + b * EB1
        pltpu.sync_copy(eu_hbm.at[pl.ds(off, EB1)], eu_s)
        pltpu.sync_copy(ei_hbm.at[pl.ds(off, EB1)], ei_s)
        for v in range(EB1 // 16):
            euv = eu_s[pl.ds(v * 16, 16)]
            eiv = ei_s[pl.ds(v * 16, 16)]
            ek = (euv * NI + eiv) ^ jnp.int32(-2147483648)
            lo = jnp.zeros((16,), _i32)
            for sh in range(11, -1, -1):
                probe = lo + ((1 << sh) - 1)
                kv = plsc.load_gather(skey_v, [probe])
                lo = lo + jnp.where(kv < ek, _i32(1 << sh), _i32(0))
            kv = plsc.load_gather(skey_v, [lo])
            pos = off + v * 16 + lax.iota(_i32, 16)
            drop = (kv == ek) | (pos >= NE)
            mu_s[pl.ds(v * 16, 16)] = jnp.where(drop, -1, euv)
            mi_s[pl.ds(v * 16, 16)] = jnp.where(drop, -1, eiv)
        pltpu.sync_copy(mu_s, mu_hbm.at[pl.ds(off, EB1)])
        pltpu.sync_copy(mi_s, mi_hbm.at[pl.ds(off, EB1)])
        return carry

    lax.fori_loop(_i32(0), _i32(E1 // EB1), block, _i32(0))


_k1 = pl.kernel(
    _k1_body,
    out_type=(jax.ShapeDtypeStruct((NE_PAD,), _i32),
              jax.ShapeDtypeStruct((NE_PAD,), _i32)),
    mesh=_mesh,
    scratch_types=[
        pltpu.VMEM((BATCH,), _i32),
        pltpu.VMEM((EB1,), _i32),
        pltpu.VMEM((EB1,), _i32),
        pltpu.VMEM((EB1,), _i32),
        pltpu.VMEM((EB1,), _i32),
    ],
    compiler_params=_sc_params,
)


NBUF = 3


def _seg_body(src_hbm, dst_hbm, table_hbm, zeros_hbm, out_hbm,
              src_s, dst_s, gidx0, gidx1, gidx2,
              sidx0, sidx1, sidx2, rows0, rows1, rows2,
              sp, gsem0, gsem1, gsem2, ssem0, ssem1, ssem2):
    c = lax.axis_index("c")
    s = lax.axis_index("s")
    gidx = (gidx0, gidx1, gidx2)
    sidx = (sidx0, sidx1, sidx2)
    rows = (rows0, rows1, rows2)
    gsem = (gsem0, gsem1, gsem2)
    ssem = (ssem0, ssem1, ssem2)
    pltpu.sync_copy(zeros_hbm, sp.at[pl.ds(s * DRAIN, DRAIN)])
    plsc.subcore_barrier()
    n_chunks = BS // CK

    def block(b, carry):
        off = s * ES + b * BS
        pltpu.sync_copy(src_hbm.at[pl.ds(off, BS)], src_s)
        pltpu.sync_copy(dst_hbm.at[pl.ds(off, BS)], dst_s)

        def make_idx(j):
            bb = j % NBUF
            for v in range(CK // 16):
                sv = src_s[pl.ds(j * CK + v * 16, 16)]
                gidx[bb][pl.ds(v * 16, 16)] = sv
                dv = dst_s[pl.ds(j * CK + v * 16, 16)]
                ld = dv - c * HALF
                owned = (ld >= 0) & (ld < HALF)
                tr = HALF + ((j * CK + v * 16 + lax.iota(_i32, 16)) & 127)
                sidx[bb][pl.ds(v * 16, 16)] = jnp.where(owned, ld, tr)

        gd = [None] * NBUF
        sd = [None] * NBUF
        for j in range(NBUF - 1):
            make_idx(j)
            bb = j % NBUF
            gd[bb] = pltpu.async_copy(table_hbm.at[gidx[bb]],
                                      rows[bb], gsem[bb])
        for j in range(n_chunks):
            bb = j % NBUF
            nj = j + NBUF - 1
            nb = nj % NBUF
            if nj < n_chunks:
                if sd[nb] is not None:
                    sd[nb].wait()
                make_idx(nj)
                gd[nb] = pltpu.async_copy(table_hbm.at[gidx[nb]],
                                          rows[nb], gsem[nb])
            gd[bb].wait()
            sd[bb] = pltpu.async_copy(rows[bb], sp.at[sidx[bb]],
                                      ssem[bb], add=True)
        for d in sd:
            if d is not None:
                d.wait()
        return carry

    lax.fori_loop(_i32(0), _i32(ES // BS), block, _i32(0))
    plsc.subcore_barrier()

    @pl.when(s < NS - 1)
    def _():
        pltpu.sync_copy(sp.at[pl.ds(s * DRAIN, DRAIN)],
                        out_hbm.at[pl.ds(c * HALF + s * DRAIN, DRAIN)])

    @pl.when(s == NS - 1)
    def _():
        rem = HALF - (NS - 1) * DRAIN
        pltpu.sync_copy(sp.at[pl.ds((NS - 1) * DRAIN, rem)],
                        out_hbm.at[pl.ds(c * HALF + (NS - 1) * DRAIN, rem)])


_seg = pl.kernel(
    _seg_body,
    out_type=jax.ShapeDtypeStruct((NU, EMB), _f32),
    mesh=_mesh,
    scratch_types=(
        [pltpu.VMEM((BS,), _i32)] * 2
        + [pltpu.VMEM((CK,), _i32)] * (2 * NBUF)
        + [pltpu.VMEM((CK, EMB), _f32)] * NBUF
        + [pltpu.VMEM_SHARED((SPR, EMB), _f32)]
        + [pltpu.SemaphoreType.DMA] * (2 * NBUF)
    ),
    compiler_params=_sc_params,
)


def _combine_chunk(idx_hbm, base, t0_hbm, t1_hbm, t2_hbm, tm_hbm,
                   out_c_hbm, out_m_hbm, idxb, bufa, bufb, bufc, bufo, gsem):
    pltpu.sync_copy(idx_hbm.at[pl.ds(base, CK)], idxb)
    pltpu.async_copy(t0_hbm.at[idxb], bufa, gsem).wait()
    pltpu.async_copy(t1_hbm.at[idxb], bufb, gsem).wait()
    pltpu.async_copy(t2_hbm.at[idxb], bufc, gsem).wait()

    def row(r, carry):
        for cv in range(EMB // 16):
            a = bufa[r, pl.ds(cv * 16, 16)]
            b = bufb[r, pl.ds(cv * 16, 16)]
            cc = bufc[r, pl.ds(cv * 16, 16)]
            t = (a + (b + cc * (1.0 / 16.0)) * (1.0 / 16.0)) * (1.0 / 3.0)
            bufo[r, pl.ds(cv * 16, 16)] = t
        return carry

    lax.fori_loop(_i32(0), _i32(CK), row, _i32(0))
    pltpu.sync_copy(bufo, out_c_hbm.at[pl.ds(base, CK)])
    pltpu.async_copy(tm_hbm.at[idxb], bufa, gsem).wait()
    pltpu.sync_copy(bufa, out_m_hbm.at[pl.ds(base, CK)])


def _k3_body(u0, u1t, u2t, umap, i0, i1t, i2t, imap, uidx, iidx,
             uc, ug, icm, ig, idxb, bufa, bufb, bufc, bufo, gsem):
    c = lax.axis_index("c")
    s = lax.axis_index("s")
    wid = s * NC + c
    _combine_chunk(uidx, wid * CK, u0, u1t, u2t, umap, uc, ug,
                   idxb, bufa, bufb, bufc, bufo, gsem)
    for q in range(POOL):
        _combine_chunk(iidx, wid * (POOL * CK) + q * CK, i0, i1t, i2t, imap,
                       icm, ig, idxb, bufa, bufb, bufc, bufo, gsem)


_k3 = pl.kernel(
    _k3_body,
    out_type=(jax.ShapeDtypeStruct((BATCH, EMB), _f32),
              jax.ShapeDtypeStruct((BATCH, EMB), _f32),
              jax.ShapeDtypeStruct((BATCH * POOL, EMB), _f32),
              jax.ShapeDtypeStruct((BATCH * POOL, EMB), _f32)),
    mesh=_mesh,
    scratch_types=[
        pltpu.VMEM((CK,), _i32),
        pltpu.VMEM((CK, EMB), _f32),
        pltpu.VMEM((CK, EMB), _f32),
        pltpu.VMEM((CK, EMB), _f32),
        pltpu.VMEM((CK, EMB), _f32),
        pltpu.SemaphoreType.DMA,
    ],
    compiler_params=_sc_params,
)

_SB = 512


def _score_body(uc_ref, ug_ref, ic_ref, ig_ref, w_ref, o_ref):
    uc = uc_ref[...]
    ug = ug_ref[...]
    w = w_ref[...]
    cols = []
    for p in range(POOL):
        sdot = jnp.sum(uc * ic_ref[p] + ug * ig_ref[p], axis=1)
        rating = 1.0 / (1.0 + jnp.exp(-sdot))
        cols.append(0.5 * w[:, p] + rating)
    o_ref[...] = jnp.stack(cols, axis=1)


def _score(uc, ug, ic3, ig3, w):
    return pl.pallas_call(
        _score_body,
        out_shape=jax.ShapeDtypeStruct((BATCH, POOL), _f32),
        grid=(BATCH // _SB,),
        in_specs=[
            pl.BlockSpec((_SB, EMB), lambda b: (b, _i32(0))),
            pl.BlockSpec((_SB, EMB), lambda b: (b, _i32(0))),
            pl.BlockSpec((POOL, _SB, EMB), lambda b: (_i32(0), b, _i32(0))),
            pl.BlockSpec((POOL, _SB, EMB), lambda b: (_i32(0), b, _i32(0))),
            pl.BlockSpec((_SB, POOL), lambda b: (b, _i32(0))),
        ],
        out_specs=pl.BlockSpec((_SB, POOL), lambda b: (b, _i32(0))),
    )(uc, ug, ic3, ig3, w)


def kernel(users, adjacent_items, items_pool, items_weight, edge_user,
           edge_item, user_preference, item_preference, user_map, item_map):
    eu = edge_user.astype(_i32)
    ei = edge_item.astype(_i32)
    pad_src = jnp.arange(NE_PAD - NE, dtype=_i32) % NU
    eu_p = jnp.concatenate([eu, pad_src])
    ei_p = jnp.concatenate([ei, pad_src])

    k_u32 = (users.astype(jnp.uint32) * jnp.uint32(NI)
             + adjacent_items.astype(jnp.uint32))
    skey = lax.bitcast_convert_type(
        jnp.sort(k_u32) ^ jnp.uint32(0x80000000), _i32)

    mu, mi = _k1(eu_p, ei_p, skey)

    up = user_preference.astype(_f32)
    ip = item_preference.astype(_f32)
    zeros = jnp.zeros((DRAIN, EMB), _f32)
    u1 = _seg(ei_p, mu, ip, zeros)
    i1 = _seg(eu_p, mi, up, zeros)
    u2 = _seg(ei_p, mu, i1, zeros)
    i2 = _seg(eu_p, mi, u1, zeros)

    uidx = users.astype(_i32)
    iidx = items_pool.astype(_i32).T.reshape(-1)
    uc, ug, icm, ig = _k3(up, u1, u2, user_map.astype(_f32),
                          ip, i1, i2, item_map.astype(_f32), uidx, iidx)

    ic3 = icm.reshape(POOL, BATCH, EMB)
    ig3 = ig.reshape(POOL, BATCH, EMB)
    return _score(uc, ug, ic3, ig3, items_weight.astype(_f32))


# speedup vs baseline: 96.3376x; 1.0212x over previous
import functools

import jax
import jax.numpy as jnp
from jax import lax
from jax.experimental import pallas as pl
from jax.experimental.pallas import tpu as pltpu
from jax.experimental.pallas import tpu_sc as plsc

NU = 50000
NI = 50000
EMB = 64
NE = 800000
NE_PAD = 819200
BATCH = 4096
POOL = 10

NC = 2
NS = 16
NW = NC * NS

E1 = NE_PAD // NW
EB1 = 1600

ES = NE_PAD // NS
BS = 2048
CK = 128

HALF = NU // 2
SPR = 25152
DRAIN = SPR // NS

_mesh = plsc.VectorSubcoreMesh(
    core_axis_name="c", subcore_axis_name="s", num_cores=NC, num_subcores=NS)
_sc_params = pltpu.CompilerParams(
    needs_layout_passes=False, use_tc_tiling_on_sc=False)

_f32 = jnp.float32
_i32 = jnp.int32


def _k1_body(eu_hbm, ei_hbm, skey_hbm, mu_hbm, mi_hbm,
             skey_v, eu_s, ei_s, mu_s, mi_s, osem_u, osem_i):
    c = lax.axis_index("c")
    s = lax.axis_index("s")
    wid = s * NC + c
    base = wid * E1
    pltpu.sync_copy(skey_hbm, skey_v)

    def block(b, carry):
        off = base + b * EB1
        pltpu.sync_copy(eu_hbm.at[pl.ds(off, EB1)], eu_s)
        pltpu.sync_copy(ei_hbm.at[pl.ds(off, EB1)], ei_s)

        @pl.when(b > 0)
        def _():
            pltpu.make_async_copy(
                mu_s, mu_hbm.at[pl.ds(off, EB1)], osem_u).wait()
            pltpu.make_async_copy(
                mi_s, mi_hbm.at[pl.ds(off, EB1)], osem_i).wait()
        for v in range(EB1 // 16):
            euv = eu_s[pl.ds(v * 16, 16)]
            eiv = ei_s[pl.ds(v * 16, 16)]
            ek = (euv * NI + eiv) ^ jnp.int32(-2147483648)
            lo = jnp.zeros((16,), _i32)
            for sh in range(11, -1, -1):
                probe = lo + ((1 << sh) - 1)
                kv = plsc.load_gather(skey_v, [probe])
                lo = lo + jnp.where(kv < ek, _i32(1 << sh), _i32(0))
            kv = plsc.load_gather(skey_v, [lo])
            pos = off + v * 16 + lax.iota(_i32, 16)
            drop = (kv == ek) | (pos >= NE)
            mu_s[pl.ds(v * 16, 16)] = jnp.where(drop, -1, euv)
            mi_s[pl.ds(v * 16, 16)] = jnp.where(drop, -1, eiv)
        pltpu.async_copy(mu_s, mu_hbm.at[pl.ds(off, EB1)], osem_u)
        pltpu.async_copy(mi_s, mi_hbm.at[pl.ds(off, EB1)], osem_i)
        return carry

    lax.fori_loop(_i32(0), _i32(E1 // EB1), block, _i32(0))
    last = base + (E1 // EB1 - 1) * EB1
    pltpu.make_async_copy(mu_s, mu_hbm.at[pl.ds(last, EB1)], osem_u).wait()
    pltpu.make_async_copy(mi_s, mi_hbm.at[pl.ds(last, EB1)], osem_i).wait()


_k1 = pl.kernel(
    _k1_body,
    out_type=(jax.ShapeDtypeStruct((NE_PAD,), _i32),
              jax.ShapeDtypeStruct((NE_PAD,), _i32)),
    mesh=_mesh,
    scratch_types=[
        pltpu.VMEM((BATCH,), _i32),
        pltpu.VMEM((EB1,), _i32),
        pltpu.VMEM((EB1,), _i32),
        pltpu.VMEM((EB1,), _i32),
        pltpu.VMEM((EB1,), _i32),
        pltpu.SemaphoreType.DMA,
        pltpu.SemaphoreType.DMA,
    ],
    compiler_params=_sc_params,
)


NBUF = 3


def _seg_body(src_hbm, dst_hbm, table_hbm, zeros_hbm, out_hbm,
              src_s, dst_s, gidx0, gidx1, gidx2,
              sidx0, sidx1, sidx2, rows0, rows1, rows2,
              sp, gsem0, gsem1, gsem2, ssem0, ssem1, ssem2):
    c = lax.axis_index("c")
    s = lax.axis_index("s")
    gidx = (gidx0, gidx1, gidx2)
    sidx = (sidx0, sidx1, sidx2)
    rows = (rows0, rows1, rows2)
    gsem = (gsem0, gsem1, gsem2)
    ssem = (ssem0, ssem1, ssem2)
    pltpu.sync_copy(zeros_hbm, sp.at[pl.ds(s * DRAIN, DRAIN)])
    plsc.subcore_barrier()
    n_chunks = BS // CK

    def block(b, carry):
        off = s * ES + b * BS
        pltpu.sync_copy(src_hbm.at[pl.ds(off, BS)], src_s)
        pltpu.sync_copy(dst_hbm.at[pl.ds(off, BS)], dst_s)

        def make_idx(j):
            bb = j % NBUF
            for v in range(CK // 16):
                sv = src_s[pl.ds(j * CK + v * 16, 16)]
                gidx[bb][pl.ds(v * 16, 16)] = sv
                dv = dst_s[pl.ds(j * CK + v * 16, 16)]
                ld = dv - c * HALF
                owned = (ld >= 0) & (ld < HALF)
                tr = HALF + ((j * CK + v * 16 + lax.iota(_i32, 16)) & 127)
                sidx[bb][pl.ds(v * 16, 16)] = jnp.where(owned, ld, tr)

        gd = [None] * NBUF
        sd = [None] * NBUF
        for j in range(NBUF - 1):
            make_idx(j)
            bb = j % NBUF
            gd[bb] = pltpu.async_copy(table_hbm.at[gidx[bb]],
                                      rows[bb], gsem[bb])
        for j in range(n_chunks):
            bb = j % NBUF
            nj = j + NBUF - 1
            nb = nj % NBUF
            if nj < n_chunks:
                if sd[nb] is not None:
                    sd[nb].wait()
                make_idx(nj)
                gd[nb] = pltpu.async_copy(table_hbm.at[gidx[nb]],
                                          rows[nb], gsem[nb])
            gd[bb].wait()
            sd[bb] = pltpu.async_copy(rows[bb], sp.at[sidx[bb]],
                                      ssem[bb], add=True)
        for d in sd:
            if d is not None:
                d.wait()
        return carry

    lax.fori_loop(_i32(0), _i32(ES // BS), block, _i32(0))
    plsc.subcore_barrier()

    @pl.when(s < NS - 1)
    def _():
        pltpu.sync_copy(sp.at[pl.ds(s * DRAIN, DRAIN)],
                        out_hbm.at[pl.ds(c * HALF + s * DRAIN, DRAIN)])

    @pl.when(s == NS - 1)
    def _():
        rem = HALF - (NS - 1) * DRAIN
        pltpu.sync_copy(sp.at[pl.ds((NS - 1) * DRAIN, rem)],
                        out_hbm.at[pl.ds(c * HALF + (NS - 1) * DRAIN, rem)])


_seg = pl.kernel(
    _seg_body,
    out_type=jax.ShapeDtypeStruct((NU, EMB), _f32),
    mesh=_mesh,
    scratch_types=(
        [pltpu.VMEM((BS,), _i32)] * 2
        + [pltpu.VMEM((CK,), _i32)] * (2 * NBUF)
        + [pltpu.VMEM((CK, EMB), _f32)] * NBUF
        + [pltpu.VMEM_SHARED((SPR, EMB), _f32)]
        + [pltpu.SemaphoreType.DMA] * (2 * NBUF)
    ),
    compiler_params=_sc_params,
)


def _combine_chunk(idx_hbm, base, t0_hbm, t1_hbm, t2_hbm, tm_hbm,
                   out_c_hbm, out_m_hbm, idxb, bufa, bufb, bufc, bufm, bufo,
                   sems):
    pltpu.sync_copy(idx_hbm.at[pl.ds(base, CK)], idxb)
    da = pltpu.async_copy(t0_hbm.at[idxb], bufa, sems[0])
    db = pltpu.async_copy(t1_hbm.at[idxb], bufb, sems[1])
    dc = pltpu.async_copy(t2_hbm.at[idxb], bufc, sems[2])
    dm = pltpu.async_copy(tm_hbm.at[idxb], bufm, sems[3])
    da.wait()
    db.wait()
    dc.wait()

    def row(r, carry):
        for cv in range(EMB // 16):
            a = bufa[r, pl.ds(cv * 16, 16)]
            b = bufb[r, pl.ds(cv * 16, 16)]
            cc = bufc[r, pl.ds(cv * 16, 16)]
            t = (a + (b + cc * (1.0 / 16.0)) * (1.0 / 16.0)) * (1.0 / 3.0)
            bufo[r, pl.ds(cv * 16, 16)] = t
        return carry

    lax.fori_loop(_i32(0), _i32(CK), row, _i32(0))
    pltpu.sync_copy(bufo, out_c_hbm.at[pl.ds(base, CK)])
    dm.wait()
    pltpu.sync_copy(bufm, out_m_hbm.at[pl.ds(base, CK)])


def _k3_body(u0, u1t, u2t, umap, i0, i1t, i2t, imap, uidx, iidx,
             uc, ug, icm, ig, idxb, bufa, bufb, bufc, bufm, bufo,
             gsem0, gsem1, gsem2, gsem3):
    c = lax.axis_index("c")
    s = lax.axis_index("s")
    wid = s * NC + c
    sems = (gsem0, gsem1, gsem2, gsem3)
    _combine_chunk(uidx, wid * CK, u0, u1t, u2t, umap, uc, ug,
                   idxb, bufa, bufb, bufc, bufm, bufo, sems)
    for q in range(POOL):
        _combine_chunk(iidx, wid * (POOL * CK) + q * CK, i0, i1t, i2t, imap,
                       icm, ig, idxb, bufa, bufb, bufc, bufm, bufo, sems)


_k3 = pl.kernel(
    _k3_body,
    out_type=(jax.ShapeDtypeStruct((BATCH, EMB), _f32),
              jax.ShapeDtypeStruct((BATCH, EMB), _f32),
              jax.ShapeDtypeStruct((BATCH * POOL, EMB), _f32),
              jax.ShapeDtypeStruct((BATCH * POOL, EMB), _f32)),
    mesh=_mesh,
    scratch_types=[
        pltpu.VMEM((CK,), _i32),
        pltpu.VMEM((CK, EMB), _f32),
        pltpu.VMEM((CK, EMB), _f32),
        pltpu.VMEM((CK, EMB), _f32),
        pltpu.VMEM((CK, EMB), _f32),
        pltpu.VMEM((CK, EMB), _f32),
        pltpu.SemaphoreType.DMA,
        pltpu.SemaphoreType.DMA,
        pltpu.SemaphoreType.DMA,
        pltpu.SemaphoreType.DMA,
    ],
    compiler_params=_sc_params,
)

_SB = 512


def _score_body(uc_ref, ug_ref, ic_ref, ig_ref, w_ref, o_ref):
    uc = uc_ref[...]
    ug = ug_ref[...]
    w = w_ref[...]
    cols = []
    for p in range(POOL):
        sdot = jnp.sum(uc * ic_ref[p] + ug * ig_ref[p], axis=1)
        rating = 1.0 / (1.0 + jnp.exp(-sdot))
        cols.append(0.5 * w[:, p] + rating)
    o_ref[...] = jnp.stack(cols, axis=1)


def _score(uc, ug, ic3, ig3, w):
    return pl.pallas_call(
        _score_body,
        out_shape=jax.ShapeDtypeStruct((BATCH, POOL), _f32),
        grid=(BATCH // _SB,),
        in_specs=[
            pl.BlockSpec((_SB, EMB), lambda b: (b, _i32(0))),
            pl.BlockSpec((_SB, EMB), lambda b: (b, _i32(0))),
            pl.BlockSpec((POOL, _SB, EMB), lambda b: (_i32(0), b, _i32(0))),
            pl.BlockSpec((POOL, _SB, EMB), lambda b: (_i32(0), b, _i32(0))),
            pl.BlockSpec((_SB, POOL), lambda b: (b, _i32(0))),
        ],
        out_specs=pl.BlockSpec((_SB, POOL), lambda b: (b, _i32(0))),
    )(uc, ug, ic3, ig3, w)


def kernel(users, adjacent_items, items_pool, items_weight, edge_user,
           edge_item, user_preference, item_preference, user_map, item_map):
    eu = edge_user.astype(_i32)
    ei = edge_item.astype(_i32)
    pad_src = jnp.arange(NE_PAD - NE, dtype=_i32) % NU
    eu_p = jnp.concatenate([eu, pad_src])
    ei_p = jnp.concatenate([ei, pad_src])

    k_u32 = (users.astype(jnp.uint32) * jnp.uint32(NI)
             + adjacent_items.astype(jnp.uint32))
    skey = lax.bitcast_convert_type(
        jnp.sort(k_u32) ^ jnp.uint32(0x80000000), _i32)

    mu, mi = _k1(eu_p, ei_p, skey)

    up = user_preference.astype(_f32)
    ip = item_preference.astype(_f32)
    zeros = jnp.zeros((DRAIN, EMB), _f32)
    u1 = _seg(ei_p, mu, ip, zeros)
    i1 = _seg(eu_p, mi, up, zeros)
    u2 = _seg(ei_p, mu, i1, zeros)
    i2 = _seg(eu_p, mi, u1, zeros)

    uidx = users.astype(_i32)
    iidx = items_pool.astype(_i32).T.reshape(-1)
    uc, ug, icm, ig = _k3(up, u1, u2, user_map.astype(_f32),
                          ip, i1, i2, item_map.astype(_f32), uidx, iidx)

    ic3 = icm.reshape(POOL, BATCH, EMB)
    ig3 = ig.reshape(POOL, BATCH, EMB)
    return _score(uc, ug, ic3, ig3, items_weight.astype(_f32))


# speedup vs baseline: 96.4815x; 1.0015x over previous
import jax
import jax.numpy as jnp
from jax import lax
from jax.experimental import pallas as pl
from jax.experimental.pallas import tpu as pltpu
from jax.experimental.pallas import tpu_sc as plsc

NU = 50000
NI = 50000
EMB = 64
NE = 800000
NE_PAD = 819200
BATCH = 4096
POOL = 10

NC = 2
NS = 16
NW = NC * NS

E1 = NE_PAD // NW
EB1 = 1600

ES = NE_PAD // NS
BS = 2048
CK = 128

HALF = NU // 2
SPR = 25152
DRAIN = SPR // NS

_mesh = plsc.VectorSubcoreMesh(
    core_axis_name="c", subcore_axis_name="s", num_cores=NC, num_subcores=NS)
_sc_params = pltpu.CompilerParams(
    needs_layout_passes=False, use_tc_tiling_on_sc=False)

_f32 = jnp.float32
_i32 = jnp.int32


def _k1_body(eu_hbm, ei_hbm, skey_hbm, mu_hbm, mi_hbm,
             skey_v, eu_s, ei_s, mu_s, mi_s, osem_u, osem_i):
    c = lax.axis_index("c")
    s = lax.axis_index("s")
    wid = s * NC + c
    base = wid * E1
    pltpu.sync_copy(skey_hbm, skey_v)

    def block(b, carry):
        off = base + b * EB1
        pltpu.sync_copy(eu_hbm.at[pl.ds(off, EB1)], eu_s)
        pltpu.sync_copy(ei_hbm.at[pl.ds(off, EB1)], ei_s)

        @pl.when(b > 0)
        def _():
            pltpu.make_async_copy(
                mu_s, mu_hbm.at[pl.ds(off, EB1)], osem_u).wait()
            pltpu.make_async_copy(
                mi_s, mi_hbm.at[pl.ds(off, EB1)], osem_i).wait()
        for v in range(EB1 // 16):
            euv = eu_s[pl.ds(v * 16, 16)]
            eiv = ei_s[pl.ds(v * 16, 16)]
            ek = (euv * NI + eiv) ^ jnp.int32(-2147483648)
            lo = jnp.zeros((16,), _i32)
            for sh in range(11, -1, -1):
                probe = lo + ((1 << sh) - 1)
                kv = plsc.load_gather(skey_v, [probe])
                lo = lo + jnp.where(kv < ek, _i32(1 << sh), _i32(0))
            kv = plsc.load_gather(skey_v, [lo])
            pos = off + v * 16 + lax.iota(_i32, 16)
            drop = (kv == ek) | (pos >= NE)
            mu_s[pl.ds(v * 16, 16)] = jnp.where(drop, -1, euv)
            mi_s[pl.ds(v * 16, 16)] = jnp.where(drop, -1, eiv)
        pltpu.async_copy(mu_s, mu_hbm.at[pl.ds(off, EB1)], osem_u)
        pltpu.async_copy(mi_s, mi_hbm.at[pl.ds(off, EB1)], osem_i)
        return carry

    lax.fori_loop(_i32(0), _i32(E1 // EB1), block, _i32(0))
    last = base + (E1 // EB1 - 1) * EB1
    pltpu.make_async_copy(mu_s, mu_hbm.at[pl.ds(last, EB1)], osem_u).wait()
    pltpu.make_async_copy(mi_s, mi_hbm.at[pl.ds(last, EB1)], osem_i).wait()


_k1 = pl.kernel(
    _k1_body,
    out_type=(jax.ShapeDtypeStruct((NE_PAD,), _i32),
              jax.ShapeDtypeStruct((NE_PAD,), _i32)),
    mesh=_mesh,
    scratch_types=[
        pltpu.VMEM((BATCH,), _i32),
        pltpu.VMEM((EB1,), _i32),
        pltpu.VMEM((EB1,), _i32),
        pltpu.VMEM((EB1,), _i32),
        pltpu.VMEM((EB1,), _i32),
        pltpu.SemaphoreType.DMA,
        pltpu.SemaphoreType.DMA,
    ],
    compiler_params=_sc_params,
)


NBUF = 3


def _seg_body(src_hbm, dst_hbm, table_hbm, zeros_hbm, out_hbm,
              src_s, dst_s, gidx0, gidx1, gidx2,
              sidx0, sidx1, sidx2, rows0, rows1, rows2,
              sp, gsem0, gsem1, gsem2, ssem0, ssem1, ssem2):
    c = lax.axis_index("c")
    s = lax.axis_index("s")
    gidx = (gidx0, gidx1, gidx2)
    sidx = (sidx0, sidx1, sidx2)
    rows = (rows0, rows1, rows2)
    gsem = (gsem0, gsem1, gsem2)
    ssem = (ssem0, ssem1, ssem2)
    pltpu.sync_copy(zeros_hbm, sp.at[pl.ds(s * DRAIN, DRAIN)])
    plsc.subcore_barrier()
    n_chunks = BS // CK

    def block(b, carry):
        off = s * ES + b * BS
        pltpu.sync_copy(src_hbm.at[pl.ds(off, BS)], src_s)
        pltpu.sync_copy(dst_hbm.at[pl.ds(off, BS)], dst_s)

        def make_idx(j):
            bb = j % NBUF
            for v in range(CK // 16):
                sv = src_s[pl.ds(j * CK + v * 16, 16)]
                gidx[bb][pl.ds(v * 16, 16)] = sv
                dv = dst_s[pl.ds(j * CK + v * 16, 16)]
                ld = dv - c * HALF
                owned = (ld >= 0) & (ld < HALF)
                tr = HALF + ((j * CK + v * 16 + lax.iota(_i32, 16)) & 127)
                sidx[bb][pl.ds(v * 16, 16)] = jnp.where(owned, ld, tr)

        gd = [None] * NBUF
        sd = [None] * NBUF
        for j in range(NBUF - 1):
            make_idx(j)
            bb = j % NBUF
            gd[bb] = pltpu.async_copy(table_hbm.at[gidx[bb]],
                                      rows[bb], gsem[bb])
        for j in range(n_chunks):
            bb = j % NBUF
            nj = j + NBUF - 1
            nb = nj % NBUF
            if nj < n_chunks:
                if sd[nb] is not None:
                    sd[nb].wait()
                make_idx(nj)
                gd[nb] = pltpu.async_copy(table_hbm.at[gidx[nb]],
                                          rows[nb], gsem[nb])
            gd[bb].wait()
            sd[bb] = pltpu.async_copy(rows[bb], sp.at[sidx[bb]],
                                      ssem[bb], add=True)
        for d in sd:
            if d is not None:
                d.wait()
        return carry

    lax.fori_loop(_i32(0), _i32(ES // BS), block, _i32(0))
    plsc.subcore_barrier()

    @pl.when(s < NS - 1)
    def _():
        pltpu.sync_copy(sp.at[pl.ds(s * DRAIN, DRAIN)],
                        out_hbm.at[pl.ds(c * HALF + s * DRAIN, DRAIN)])

    @pl.when(s == NS - 1)
    def _():
        rem = HALF - (NS - 1) * DRAIN
        pltpu.sync_copy(sp.at[pl.ds((NS - 1) * DRAIN, rem)],
                        out_hbm.at[pl.ds(c * HALF + (NS - 1) * DRAIN, rem)])


_seg = pl.kernel(
    _seg_body,
    out_type=jax.ShapeDtypeStruct((NU, EMB), _f32),
    mesh=_mesh,
    scratch_types=(
        [pltpu.VMEM((BS,), _i32)] * 2
        + [pltpu.VMEM((CK,), _i32)] * (2 * NBUF)
        + [pltpu.VMEM((CK, EMB), _f32)] * NBUF
        + [pltpu.VMEM_SHARED((SPR, EMB), _f32)]
        + [pltpu.SemaphoreType.DMA] * (2 * NBUF)
    ),
    compiler_params=_sc_params,
)


def _combine_chunk(idx_hbm, base, t0_hbm, t1_hbm, t2_hbm, tm_hbm,
                   out_c_hbm, out_m_hbm, idxb, bufa, bufb, bufc, bufm, bufo,
                   sems):
    pltpu.sync_copy(idx_hbm.at[pl.ds(base, CK)], idxb)
    da = pltpu.async_copy(t0_hbm.at[idxb], bufa, sems[0])
    db = pltpu.async_copy(t1_hbm.at[idxb], bufb, sems[1])
    dc = pltpu.async_copy(t2_hbm.at[idxb], bufc, sems[2])
    dm = pltpu.async_copy(tm_hbm.at[idxb], bufm, sems[3])
    da.wait()
    db.wait()
    dc.wait()

    def row(r, carry):
        for cv in range(EMB // 16):
            a = bufa[r, pl.ds(cv * 16, 16)]
            b = bufb[r, pl.ds(cv * 16, 16)]
            cc = bufc[r, pl.ds(cv * 16, 16)]
            t = (a + (b + cc * (1.0 / 16.0)) * (1.0 / 16.0)) * (1.0 / 3.0)
            bufo[r, pl.ds(cv * 16, 16)] = t
        return carry

    lax.fori_loop(_i32(0), _i32(CK), row, _i32(0))
    pltpu.sync_copy(bufo, out_c_hbm.at[pl.ds(base, CK)])
    dm.wait()
    pltpu.sync_copy(bufm, out_m_hbm.at[pl.ds(base, CK)])


def _k3_body(u0, u1t, u2t, umap, i0, i1t, i2t, imap, uidx, iidx,
             uc, ug, icm, ig, idxb, bufa, bufb, bufc, bufm, bufo,
             gsem0, gsem1, gsem2, gsem3):
    c = lax.axis_index("c")
    s = lax.axis_index("s")
    wid = s * NC + c
    sems = (gsem0, gsem1, gsem2, gsem3)
    _combine_chunk(uidx, wid * CK, u0, u1t, u2t, umap, uc, ug,
                   idxb, bufa, bufb, bufc, bufm, bufo, sems)
    for q in range(POOL):
        _combine_chunk(iidx, wid * (POOL * CK) + q * CK, i0, i1t, i2t, imap,
                       icm, ig, idxb, bufa, bufb, bufc, bufm, bufo, sems)


_k3 = pl.kernel(
    _k3_body,
    out_type=(jax.ShapeDtypeStruct((BATCH, EMB), _f32),
              jax.ShapeDtypeStruct((BATCH, EMB), _f32),
              jax.ShapeDtypeStruct((BATCH * POOL, EMB), _f32),
              jax.ShapeDtypeStruct((BATCH * POOL, EMB), _f32)),
    mesh=_mesh,
    scratch_types=[
        pltpu.VMEM((CK,), _i32),
        pltpu.VMEM((CK, EMB), _f32),
        pltpu.VMEM((CK, EMB), _f32),
        pltpu.VMEM((CK, EMB), _f32),
        pltpu.VMEM((CK, EMB), _f32),
        pltpu.VMEM((CK, EMB), _f32),
        pltpu.SemaphoreType.DMA,
        pltpu.SemaphoreType.DMA,
        pltpu.SemaphoreType.DMA,
        pltpu.SemaphoreType.DMA,
    ],
    compiler_params=_sc_params,
)

_SB = 512


def _score_body(uc_ref, ug_ref, ic_ref, ig_ref, w_ref, o_ref):
    uc = uc_ref[...]
    ug = ug_ref[...]
    w = w_ref[...]
    cols = []
    for p in range(POOL):
        sdot = jnp.sum(uc * ic_ref[p] + ug * ig_ref[p], axis=1)
        rating = 1.0 / (1.0 + jnp.exp(-sdot))
        cols.append(0.5 * w[:, p] + rating)
    o_ref[...] = jnp.stack(cols, axis=1)


def _score(uc, ug, ic3, ig3, w):
    return pl.pallas_call(
        _score_body,
        out_shape=jax.ShapeDtypeStruct((BATCH, POOL), _f32),
        grid=(BATCH // _SB,),
        in_specs=[
            pl.BlockSpec((_SB, EMB), lambda b: (b, _i32(0))),
            pl.BlockSpec((_SB, EMB), lambda b: (b, _i32(0))),
            pl.BlockSpec((POOL, _SB, EMB), lambda b: (_i32(0), b, _i32(0))),
            pl.BlockSpec((POOL, _SB, EMB), lambda b: (_i32(0), b, _i32(0))),
            pl.BlockSpec((_SB, POOL), lambda b: (b, _i32(0))),
        ],
        out_specs=pl.BlockSpec((_SB, POOL), lambda b: (b, _i32(0))),
    )(uc, ug, ic3, ig3, w)


def kernel(users, adjacent_items, items_pool, items_weight, edge_user,
           edge_item, user_preference, item_preference, user_map, item_map):
    eu = edge_user.astype(_i32)
    ei = edge_item.astype(_i32)
    pad_src = jnp.arange(NE_PAD - NE, dtype=_i32) % NU
    eu_p = jnp.concatenate([eu, pad_src])
    ei_p = jnp.concatenate([ei, pad_src])

    k_u32 = (users.astype(jnp.uint32) * jnp.uint32(NI)
             + adjacent_items.astype(jnp.uint32))
    skey = lax.bitcast_convert_type(
        jnp.sort(k_u32) ^ jnp.uint32(0x80000000), _i32)

    mu, mi = _k1(eu_p, ei_p, skey)

    up = user_preference.astype(_f32)
    ip = item_preference.astype(_f32)
    zeros = jnp.zeros((DRAIN, EMB), _f32)
    u1 = _seg(ei_p, mu, ip, zeros)
    i1 = _seg(eu_p, mi, up, zeros)
    u2 = _seg(ei_p, mu, i1, zeros)
    i2 = _seg(eu_p, mi, u1, zeros)

    uidx = users.astype(_i32)
    iidx = items_pool.astype(_i32).T.reshape(-1)
    uc, ug, icm, ig = _k3(up, u1, u2, user_map.astype(_f32),
                          ip, i1, i2, item_map.astype(_f32), uidx, iidx)

    ic3 = icm.reshape(POOL, BATCH, EMB)
    ig3 = ig.reshape(POOL, BATCH, EMB)
    return _score(uc, ug, ic3, ig3, items_weight.astype(_f32))
